# baseline (speedup 1.0000x reference)
import functools

import jax
import jax.numpy as jnp
from jax import lax
from jax.experimental import pallas as pl
from jax.experimental.pallas import tpu as pltpu
from jax.experimental.pallas import tpu_sc as plsc

VOCAB = 100000
EMBED = 64
WINDOW = 20
HIDDEN = 128

BV = 5000
NB = VOCAB // BV


_IDX_PAD = 32


@functools.cache
def _get_sc_gather():
    mesh = plsc.VectorSubcoreMesh(core_axis_name="c", subcore_axis_name="s")

    @functools.partial(
        pl.kernel,
        out_type=jax.ShapeDtypeStruct((WINDOW, EMBED), jnp.float32),
        mesh=mesh,
        scratch_types=[
            pltpu.VMEM((_IDX_PAD,), jnp.int32),
            pltpu.VMEM((WINDOW, EMBED), jnp.float32),
            pltpu.SemaphoreType.DMA,
        ],
        compiler_params=pltpu.CompilerParams(needs_layout_passes=False),
    )
    def _sc_gather(idx_hbm, emb_hbm, out_hbm, idx_v, sel_v, sem):
        c = lax.axis_index("c")
        s = lax.axis_index("s")

        @pl.when(jnp.logical_and(c == 0, s == 0))
        def _():
            pltpu.sync_copy(idx_hbm, idx_v.at[pl.ds(0, WINDOW)])
            lane = lax.iota(jnp.int32, 16)
            copies = []
            for r in range(WINDOW):
                chunk = idx_v[pl.ds((r // 16) * 16, 16)]
                xr = jnp.max(jnp.where(lane == (r % 16), chunk,
                                       jnp.zeros((16,), jnp.int32)))
                copies.append(pltpu.async_copy(
                    emb_hbm.at[pl.ds(xr, 1), :],
                    sel_v.at[pl.ds(r, 1), :],
                    sem,
                ))
            for cp in copies:
                cp.wait()
            pltpu.sync_copy(sel_v, out_hbm)

    return _sc_gather


_NT = (((1,), (1,)), ((), ()))


def _mlp_body(g_hbm, w1_hbm, b1_hbm, w2_ref, b2_hbm, out_hbm,
              g_ref, w1_ref, b1_ref, h_ref, lg_ref, b2s_ref, m_ref, s_ref,
              sem_g, sem_w1, sem_b1, sem_b2, sem_out):
    j = pl.program_id(0)

    @pl.when(j == 0)
    def _():
        cg = pltpu.make_async_copy(g_hbm, g_ref, sem_g)
        cw = pltpu.make_async_copy(w1_hbm, w1_ref, sem_w1)
        cb = pltpu.make_async_copy(b1_hbm, b1_ref, sem_b1)
        c2 = pltpu.make_async_copy(b2_hbm, b2s_ref, sem_b2)
        cg.start(); cw.start(); cb.start(); c2.start()
        cg.wait(); cw.wait(); cb.wait(); c2.wait()
        z1 = lax.dot_general(g_ref[:], w1_ref[:], _NT,
                             preferred_element_type=jnp.float32)
        h_ref[:] = jnp.maximum(z1 + b1_ref[:], 0.0)

    z = (lax.dot_general(h_ref[:], w2_ref[:], _NT,
                         preferred_element_type=jnp.float32)
         + b2s_ref[pl.ds(j, 1), :])
    lg_ref[pl.ds(j, 1), :] = z

    bm = jnp.max(z, axis=1, keepdims=True)

    @pl.when(j == 0)
    def _():
        m_ref[:] = bm
        s_ref[:] = jnp.sum(jnp.exp(z - bm), axis=1, keepdims=True)

    @pl.when(j > 0)
    def _():
        m_old = m_ref[:]
        m_new = jnp.maximum(m_old, bm)
        s_ref[:] = (s_ref[:] * jnp.exp(m_old - m_new)
                    + jnp.sum(jnp.exp(z - m_new), axis=1, keepdims=True))
        m_ref[:] = m_new

    @pl.when(j == NB - 1)
    def _():
        lg_ref[:] = lg_ref[:] - (m_ref[:] + jnp.log(s_ref[:]))
        co = pltpu.make_async_copy(lg_ref, out_hbm, sem_out)
        co.start()
        co.wait()


_HBM = pltpu.MemorySpace.HBM

_mlp_call = pl.pallas_call(
    _mlp_body,
    grid=(NB,),
    in_specs=[
        pl.BlockSpec(memory_space=_HBM),
        pl.BlockSpec(memory_space=_HBM),
        pl.BlockSpec(memory_space=_HBM),
        pl.BlockSpec((BV, HIDDEN), lambda j: (j, 0)),
        pl.BlockSpec(memory_space=_HBM),
    ],
    out_specs=pl.BlockSpec(memory_space=_HBM),
    out_shape=jax.ShapeDtypeStruct((NB, BV), jnp.float32),
    scratch_shapes=[
        pltpu.VMEM((1, WINDOW * EMBED), jnp.float32),
        pltpu.VMEM((HIDDEN, WINDOW * EMBED), jnp.float32),
        pltpu.VMEM((1, HIDDEN), jnp.float32),
        pltpu.VMEM((1, HIDDEN), jnp.float32),
        pltpu.VMEM((NB, BV), jnp.float32),
        pltpu.VMEM((NB, BV), jnp.float32),
        pltpu.VMEM((1, 1), jnp.float32),
        pltpu.VMEM((1, 1), jnp.float32),
        pltpu.SemaphoreType.DMA,
        pltpu.SemaphoreType.DMA,
        pltpu.SemaphoreType.DMA,
        pltpu.SemaphoreType.DMA,
        pltpu.SemaphoreType.DMA,
    ],
    compiler_params=pltpu.CompilerParams(
        dimension_semantics=("arbitrary",),
    ),
)


def kernel(x, emb, W1, b1, W2, b2):
    g = _get_sc_gather()(x.astype(jnp.int32), emb)
    out = _mlp_call(
        g.reshape(1, WINDOW * EMBED),
        W1,
        b1.reshape(1, HIDDEN),
        W2,
        b2.reshape(NB, BV),
    )
    return out.reshape(1, VOCAB)


# speedup vs baseline: 1.5338x; 1.5338x over previous
import functools

import jax
import jax.numpy as jnp
from jax import lax
from jax.experimental import pallas as pl
from jax.experimental.pallas import tpu as pltpu
from jax.experimental.pallas import tpu_sc as plsc

VOCAB = 100000
EMBED = 64
WINDOW = 20
HIDDEN = 128

BV = 4000
NB = VOCAB // BV


_IDX_PAD = 32


@functools.cache
def _get_sc_gather():
    mesh = plsc.VectorSubcoreMesh(core_axis_name="c", subcore_axis_name="s")

    @functools.partial(
        pl.kernel,
        out_type=jax.ShapeDtypeStruct((WINDOW, EMBED, 128), jnp.float32),
        mesh=mesh,
        scratch_types=[
            pltpu.VMEM((_IDX_PAD,), jnp.int32),
            pltpu.VMEM((EMBED, 128), jnp.float32),
            pltpu.SemaphoreType.DMA,
            pltpu.SemaphoreType.DMA,
        ],
        compiler_params=pltpu.CompilerParams(needs_layout_passes=False),
    )
    def _sc_gather(idx_hbm, embt_hbm, out_hbm, idx_v, slab_v, sem, sem2):
        c = lax.axis_index("c")
        s = lax.axis_index("s")
        wid = s * 2 + c

        @pl.when(wid < WINDOW)
        def _():
            pltpu.sync_copy(idx_hbm, idx_v.at[pl.ds(0, WINDOW)])
            lane = lax.iota(jnp.int32, 16)
            c0 = idx_v[pl.ds(0, 16)]
            c1 = idx_v[pl.ds(16, 16)]
            chunk = jnp.where(jnp.full((16,), wid, jnp.int32)
                              < jnp.full((16,), 16, jnp.int32), c0, c1)
            xr = jnp.max(jnp.where(lane == (wid & 15), chunk,
                                   jnp.zeros((16,), jnp.int32)))
            base = pl.multiple_of((xr >> 7) * 128, 128)
            pltpu.async_copy(
                embt_hbm.at[:, pl.ds(base, 128)], slab_v, sem,
            ).wait()
            pltpu.async_copy(slab_v, out_hbm.at[wid], sem2).wait()

    return _sc_gather


_NT = (((1,), (1,)), ((), ()))


def _mlp_body(slab_hbm, subs_hbm, w1_hbm, b1_hbm, w2_ref, b2_hbm, out_hbm,
              slab_ref, subs_ref, w1_ref, b1_ref, h_ref, lg_ref, b2s_ref,
              m_ref, s_ref, sem_sl, sem_su, sem_w1, sem_b1, sem_b2, sem_out):
    j = pl.program_id(0)

    @pl.when(j == 0)
    def _():
        cs = pltpu.make_async_copy(slab_hbm, slab_ref, sem_sl)
        cu = pltpu.make_async_copy(subs_hbm, subs_ref, sem_su)
        cw = pltpu.make_async_copy(w1_hbm, w1_ref, sem_w1)
        cb = pltpu.make_async_copy(b1_hbm, b1_ref, sem_b1)
        c2 = pltpu.make_async_copy(b2_hbm, b2s_ref, sem_b2)
        cs.start(); cu.start(); cw.start(); cb.start(); c2.start()
        cs.wait(); cu.wait(); cw.wait(); cb.wait(); c2.wait()
        lanes = lax.broadcasted_iota(jnp.int32, (WINDOW, EMBED, 128), 2)
        gsel = jnp.sum(jnp.where(lanes == subs_ref[:], slab_ref[:], 0.0),
                       axis=2)
        z1 = jnp.zeros((1, HIDDEN), jnp.float32)
        for r in range(WINDOW):
            z1 = z1 + lax.dot_general(
                gsel[r:r + 1, :], w1_ref[:, r * EMBED:(r + 1) * EMBED], _NT,
                preferred_element_type=jnp.float32)
        h_ref[:] = jnp.maximum(z1 + b1_ref[:], 0.0)

    z = (lax.dot_general(h_ref[:], w2_ref[:], _NT,
                         preferred_element_type=jnp.float32)
         + b2s_ref[pl.ds(j, 1), :])
    lg_ref[pl.ds(j, 1), :] = z

    bm = jnp.max(z, axis=1, keepdims=True)

    @pl.when(j == 0)
    def _():
        m_ref[:] = bm
        s_ref[:] = jnp.sum(jnp.exp(z - bm), axis=1, keepdims=True)

    @pl.when(j > 0)
    def _():
        m_old = m_ref[:]
        m_new = jnp.maximum(m_old, bm)
        s_ref[:] = (s_ref[:] * jnp.exp(m_old - m_new)
                    + jnp.sum(jnp.exp(z - m_new), axis=1, keepdims=True))
        m_ref[:] = m_new

    @pl.when(j == NB - 1)
    def _():
        lg_ref[:] = lg_ref[:] - (m_ref[:] + jnp.log(s_ref[:]))
        co = pltpu.make_async_copy(lg_ref, out_hbm, sem_out)
        co.start()
        co.wait()


_HBM = pltpu.MemorySpace.HBM

_mlp_call = pl.pallas_call(
    _mlp_body,
    grid=(NB,),
    in_specs=[
        pl.BlockSpec(memory_space=_HBM),
        pl.BlockSpec(memory_space=_HBM),
        pl.BlockSpec(memory_space=_HBM),
        pl.BlockSpec(memory_space=_HBM),
        pl.BlockSpec((BV, HIDDEN), lambda j: (j, 0)),
        pl.BlockSpec(memory_space=_HBM),
    ],
    out_specs=pl.BlockSpec(memory_space=_HBM),
    out_shape=jax.ShapeDtypeStruct((NB, BV), jnp.float32),
    scratch_shapes=[
        pltpu.VMEM((WINDOW, EMBED, 128), jnp.float32),
        pltpu.VMEM((WINDOW, 1, 1), jnp.int32),
        pltpu.VMEM((HIDDEN, WINDOW * EMBED), jnp.float32),
        pltpu.VMEM((1, HIDDEN), jnp.float32),
        pltpu.VMEM((1, HIDDEN), jnp.float32),
        pltpu.VMEM((NB, BV), jnp.float32),
        pltpu.VMEM((NB, BV), jnp.float32),
        pltpu.VMEM((1, 1), jnp.float32),
        pltpu.VMEM((1, 1), jnp.float32),
        pltpu.SemaphoreType.DMA,
        pltpu.SemaphoreType.DMA,
        pltpu.SemaphoreType.DMA,
        pltpu.SemaphoreType.DMA,
        pltpu.SemaphoreType.DMA,
        pltpu.SemaphoreType.DMA,
    ],
    compiler_params=pltpu.CompilerParams(
        dimension_semantics=("arbitrary",),
    ),
)


def kernel(x, emb, W1, b1, W2, b2):
    xi = x.astype(jnp.int32)
    slabs = _get_sc_gather()(xi, emb.T)
    subs = (xi & 127).reshape(WINDOW, 1, 1)
    out = _mlp_call(
        slabs,
        subs,
        W1,
        b1.reshape(1, HIDDEN),
        W2,
        b2.reshape(NB, BV),
    )
    return out.reshape(1, VOCAB)


# speedup vs baseline: 1.6495x; 1.0755x over previous
import functools

import jax
import jax.numpy as jnp
from jax import lax
from jax.experimental import pallas as pl
from jax.experimental.pallas import tpu as pltpu
from jax.experimental.pallas import tpu_sc as plsc

VOCAB = 100000
EMBED = 64
WINDOW = 20
HIDDEN = 128

BV = 5000
NB = VOCAB // BV


_IDX_PAD = 32


@functools.cache
def _get_sc_gather():
    mesh = plsc.VectorSubcoreMesh(core_axis_name="c", subcore_axis_name="s")

    @functools.partial(
        pl.kernel,
        out_type=jax.ShapeDtypeStruct((WINDOW, EMBED, 128), jnp.float32),
        mesh=mesh,
        scratch_types=[
            pltpu.VMEM((_IDX_PAD,), jnp.int32),
            pltpu.VMEM((EMBED, 128), jnp.float32),
            pltpu.SemaphoreType.DMA,
            pltpu.SemaphoreType.DMA,
        ],
        compiler_params=pltpu.CompilerParams(needs_layout_passes=False),
    )
    def _sc_gather(idx_hbm, embt_hbm, out_hbm, idx_v, slab_v, sem, sem2):
        c = lax.axis_index("c")
        s = lax.axis_index("s")
        wid = s * 2 + c

        @pl.when(wid < WINDOW)
        def _():
            pltpu.sync_copy(idx_hbm, idx_v.at[pl.ds(0, WINDOW)])
            lane = lax.iota(jnp.int32, 16)
            c0 = idx_v[pl.ds(0, 16)]
            c1 = idx_v[pl.ds(16, 16)]
            chunk = jnp.where(jnp.full((16,), wid, jnp.int32)
                              < jnp.full((16,), 16, jnp.int32), c0, c1)
            xr = jnp.max(jnp.where(lane == (wid & 15), chunk,
                                   jnp.zeros((16,), jnp.int32)))
            base = pl.multiple_of((xr >> 7) * 128, 128)
            pltpu.async_copy(
                embt_hbm.at[:, pl.ds(base, 128)], slab_v, sem,
            ).wait()
            pltpu.async_copy(slab_v, out_hbm.at[wid], sem2).wait()

    return _sc_gather


_NT = (((1,), (1,)), ((), ()))


def _mlp_body(slab_hbm, subs_hbm, w1_hbm, b1_hbm, w2_ref, b2_hbm, out_hbm,
              slab_ref, subs_ref, w1_ref, b1_ref, h_ref, lg_ref, b2s_ref,
              m_ref, s_ref, sem_sl, sem_su, sem_w1, sem_b1, sem_b2, sem_out):
    j = pl.program_id(0)

    @pl.when(j == 0)
    def _():
        cs = pltpu.make_async_copy(slab_hbm, slab_ref, sem_sl)
        cu = pltpu.make_async_copy(subs_hbm, subs_ref, sem_su)
        cw = pltpu.make_async_copy(w1_hbm, w1_ref, sem_w1)
        cb = pltpu.make_async_copy(b1_hbm, b1_ref, sem_b1)
        c2 = pltpu.make_async_copy(b2_hbm, b2s_ref, sem_b2)
        cs.start(); cu.start(); cw.start(); cb.start(); c2.start()
        cs.wait(); cu.wait(); cw.wait(); cb.wait(); c2.wait()
        lanes = lax.broadcasted_iota(jnp.int32, (WINDOW, EMBED, 128), 2)
        gsel = jnp.sum(jnp.where(lanes == subs_ref[:], slab_ref[:], 0.0),
                       axis=2)
        z1 = jnp.zeros((1, HIDDEN), jnp.float32)
        for r in range(WINDOW):
            z1 = z1 + lax.dot_general(
                gsel[r:r + 1, :], w1_ref[:, r * EMBED:(r + 1) * EMBED], _NT,
                preferred_element_type=jnp.float32)
        h_ref[:] = jnp.maximum(z1 + b1_ref[:], 0.0)

    z = (lax.dot_general(h_ref[:], w2_ref[:], _NT,
                         preferred_element_type=jnp.float32)
         + b2s_ref[pl.ds(j, 1), :])
    lg_ref[pl.ds(j, 1), :] = z

    bm = jnp.max(z, axis=1, keepdims=True)

    @pl.when(j == 0)
    def _():
        m_ref[:] = bm
        s_ref[:] = jnp.sum(jnp.exp(z - bm), axis=1, keepdims=True)

    @pl.when(j > 0)
    def _():
        m_old = m_ref[:]
        m_new = jnp.maximum(m_old, bm)
        s_ref[:] = (s_ref[:] * jnp.exp(m_old - m_new)
                    + jnp.sum(jnp.exp(z - m_new), axis=1, keepdims=True))
        m_ref[:] = m_new

    @pl.when(j == NB - 1)
    def _():
        lg_ref[:] = lg_ref[:] - (m_ref[:] + jnp.log(s_ref[:]))
        co = pltpu.make_async_copy(lg_ref, out_hbm, sem_out)
        co.start()
        co.wait()


_HBM = pltpu.MemorySpace.HBM

_mlp_call = pl.pallas_call(
    _mlp_body,
    grid=(NB,),
    in_specs=[
        pl.BlockSpec(memory_space=_HBM),
        pl.BlockSpec(memory_space=_HBM),
        pl.BlockSpec(memory_space=_HBM),
        pl.BlockSpec(memory_space=_HBM),
        pl.BlockSpec((BV, HIDDEN), lambda j: (j, 0)),
        pl.BlockSpec(memory_space=_HBM),
    ],
    out_specs=pl.BlockSpec(memory_space=_HBM),
    out_shape=jax.ShapeDtypeStruct((NB, BV), jnp.float32),
    scratch_shapes=[
        pltpu.VMEM((WINDOW, EMBED, 128), jnp.float32),
        pltpu.VMEM((WINDOW, 1, 1), jnp.int32),
        pltpu.VMEM((HIDDEN, WINDOW * EMBED), jnp.float32),
        pltpu.VMEM((1, HIDDEN), jnp.float32),
        pltpu.VMEM((1, HIDDEN), jnp.float32),
        pltpu.VMEM((NB, BV), jnp.float32),
        pltpu.VMEM((NB, BV), jnp.float32),
        pltpu.VMEM((1, 1), jnp.float32),
        pltpu.VMEM((1, 1), jnp.float32),
        pltpu.SemaphoreType.DMA,
        pltpu.SemaphoreType.DMA,
        pltpu.SemaphoreType.DMA,
        pltpu.SemaphoreType.DMA,
        pltpu.SemaphoreType.DMA,
        pltpu.SemaphoreType.DMA,
    ],
    compiler_params=pltpu.CompilerParams(
        dimension_semantics=("arbitrary",),
    ),
)


def kernel(x, emb, W1, b1, W2, b2):
    xi = x.astype(jnp.int32)
    slabs = _get_sc_gather()(xi, emb.T)
    subs = (xi & 127).reshape(WINDOW, 1, 1)
    out = _mlp_call(
        slabs,
        subs,
        W1,
        b1.reshape(1, HIDDEN),
        W2,
        b2.reshape(NB, BV),
    )
    return out.reshape(1, VOCAB)
